# baseline (speedup 1.0000x reference)
import functools

import jax
import jax.numpy as jnp
from jax.experimental import pallas as pl
from jax.experimental.pallas import tpu as pltpu

_N_BINS = 10


def _ece_body(x_ref, lab_ref, bounds_smem, out_ref, zs_ref, *,
              n_total, n_classes):
    i = pl.program_id(0)
    nsteps = pl.num_programs(0)

    @pl.when(i == 0)
    def _init():
        zs_ref[...] = jnp.zeros_like(zs_ref)

    x = x_ref[...]
    lab = lab_ref[...].astype(jnp.int32)
    tn = x.shape[0]

    e = jnp.exp(x)
    rinv = 1.0 / jnp.sum(e, axis=1, keepdims=True)
    p = e * rinv

    iota_c = jax.lax.broadcasted_iota(jnp.int32, (tn, n_classes), 1)
    z = jnp.where(lab == iota_c, p - 1.0, p)

    zs_ref[0:1, :] += jnp.sum(z, axis=0, keepdims=True)
    for b in range(1, _N_BINS):
        t = bounds_smem[0, b]
        zb = jnp.where(p > t, z, 0.0)
        zs_ref[b : b + 1, :] += jnp.sum(zb, axis=0, keepdims=True)

    @pl.when(i == nsteps - 1)
    def _fin():
        zs = zs_ref[...]
        d = zs[0:_N_BINS, :] - zs[1 : _N_BINS + 1, :]
        sce = jnp.sum(jnp.abs(d)) / float(n_total * n_classes)
        out_ref[...] = sce[None, None]


def kernel(logits, labels):
    n, c = logits.shape
    tn = 4000
    assert n % tn == 0
    lab2 = labels.astype(jnp.int8).reshape(n, 1)
    bounds = jnp.linspace(0.0, 1.0, _N_BINS + 1).astype(jnp.float32)
    bounds2 = bounds.reshape(1, _N_BINS + 1)

    body = functools.partial(_ece_body, n_total=n, n_classes=c)

    out = pl.pallas_call(
        body,
        grid=(n // tn,),
        in_specs=[
            pl.BlockSpec((tn, c), lambda i: (i, 0)),
            pl.BlockSpec((tn, 1), lambda i: (i, 0)),
            pl.BlockSpec(memory_space=pltpu.SMEM),
        ],
        out_specs=pl.BlockSpec((1, 1), lambda i: (0, 0)),
        scratch_shapes=[
            pltpu.VMEM((_N_BINS + 1, c), jnp.float32),
        ],
        out_shape=jax.ShapeDtypeStruct((1, 1), jnp.float32),
        compiler_params=pltpu.CompilerParams(
            dimension_semantics=("arbitrary",)),
    )(logits, lab2, bounds2)
    return out.reshape(-1)


# speedup vs baseline: 1.1374x; 1.1374x over previous
import functools

import jax
import jax.numpy as jnp
from jax.experimental import pallas as pl
from jax.experimental.pallas import tpu as pltpu

_N_BINS = 10


def _ece_body(x_ref, lab_ref, bounds_smem, out_ref, zs_ref, *,
              n_total, n_classes):
    i = pl.program_id(0)
    nsteps = pl.num_programs(0)

    @pl.when(i == 0)
    def _init():
        zs_ref[...] = jnp.zeros_like(zs_ref)

    x = x_ref[...]
    lab = lab_ref[...]
    tn = x.shape[0]

    e = jnp.exp(x)
    ones_c = jnp.ones((n_classes, 1), jnp.float32)
    rowsum = jax.lax.dot_general(e, ones_c, (((1,), (0,)), ((), ())),
                                 preferred_element_type=jnp.float32)
    rinv = 1.0 / rowsum
    p = e * rinv

    iota_c = jax.lax.broadcasted_iota(jnp.int32, (tn, n_classes), 1)
    z = jnp.where(lab == iota_c, p - 1.0, p)

    ones_r = jnp.ones((1, tn), jnp.float32)
    zs_ref[0:1, :] += jax.lax.dot_general(
        ones_r, z, (((1,), (0,)), ((), ())),
        preferred_element_type=jnp.float32)
    for b in range(1, _N_BINS):
        t = bounds_smem[0, b]
        zb = jnp.where(p > t, z, 0.0)
        if b % 2 == 1:
            zs_ref[b : b + 1, :] += jax.lax.dot_general(
                ones_r, zb, (((1,), (0,)), ((), ())),
                preferred_element_type=jnp.float32)
        else:
            zs_ref[b : b + 1, :] += jnp.sum(zb, axis=0, keepdims=True)

    @pl.when(i == nsteps - 1)
    def _fin():
        zs = zs_ref[...]
        d = zs[0:_N_BINS, :] - zs[1 : _N_BINS + 1, :]
        sce = jnp.sum(jnp.abs(d)) / float(n_total * n_classes)
        out_ref[...] = sce[None, None]


def kernel(logits, labels):
    n, c = logits.shape
    tn = 4000
    assert n % tn == 0
    lab2 = labels.astype(jnp.int32).reshape(n, 1)
    bounds = jnp.linspace(0.0, 1.0, _N_BINS + 1).astype(jnp.float32)
    bounds2 = bounds.reshape(1, _N_BINS + 1)

    body = functools.partial(_ece_body, n_total=n, n_classes=c)

    out = pl.pallas_call(
        body,
        grid=(n // tn,),
        in_specs=[
            pl.BlockSpec((tn, c), lambda i: (i, 0)),
            pl.BlockSpec((tn, 1), lambda i: (i, 0)),
            pl.BlockSpec(memory_space=pltpu.SMEM),
        ],
        out_specs=pl.BlockSpec((1, 1), lambda i: (0, 0)),
        scratch_shapes=[
            pltpu.VMEM((_N_BINS + 1, c), jnp.float32),
        ],
        out_shape=jax.ShapeDtypeStruct((1, 1), jnp.float32),
        compiler_params=pltpu.CompilerParams(
            dimension_semantics=("arbitrary",)),
    )(logits, lab2, bounds2)
    return out.reshape(-1)


# speedup vs baseline: 1.1720x; 1.0304x over previous
import functools

import jax
import jax.numpy as jnp
from jax.experimental import pallas as pl
from jax.experimental.pallas import tpu as pltpu

_N_BINS = 10


def _ece_body(x_ref, lab_ref, bounds_smem, out_ref, zs_ref, *,
              n_total, n_classes):
    i = pl.program_id(0)
    nsteps = pl.num_programs(0)

    @pl.when(i == 0)
    def _init():
        zs_ref[...] = jnp.zeros_like(zs_ref)

    x = x_ref[...]
    lab = lab_ref[...]
    tn = x.shape[0]

    e = jnp.exp(x)
    ones_c = jnp.ones((n_classes, 1), jnp.float32)
    rowsum = jax.lax.dot_general(e, ones_c, (((1,), (0,)), ((), ())),
                                 preferred_element_type=jnp.float32)
    rinv = 1.0 / rowsum
    p = e * rinv

    iota_c = jax.lax.broadcasted_iota(jnp.int32, (tn, n_classes), 1)
    z = jnp.where(lab == iota_c, p - 1.0, p)

    ones_r = jnp.ones((1, tn), jnp.float32)
    zs_ref[0:1, :] += jax.lax.dot_general(
        ones_r, z, (((1,), (0,)), ((), ())),
        preferred_element_type=jnp.float32)
    for b in range(1, _N_BINS):
        t = bounds_smem[0, b]
        zb = jnp.where(p > t, z, 0.0)
        zs_ref[b : b + 1, :] += jax.lax.dot_general(
            ones_r, zb, (((1,), (0,)), ((), ())),
            preferred_element_type=jnp.float32)

    @pl.when(i == nsteps - 1)
    def _fin():
        zs = zs_ref[...]
        d = zs[0:_N_BINS, :] - zs[1 : _N_BINS + 1, :]
        sce = jnp.sum(jnp.abs(d)) / float(n_total * n_classes)
        out_ref[...] = sce[None, None]


def kernel(logits, labels):
    n, c = logits.shape
    tn = 4000
    assert n % tn == 0
    lab2 = labels.astype(jnp.int32).reshape(n, 1)
    bounds = jnp.linspace(0.0, 1.0, _N_BINS + 1).astype(jnp.float32)
    bounds2 = bounds.reshape(1, _N_BINS + 1)

    body = functools.partial(_ece_body, n_total=n, n_classes=c)

    out = pl.pallas_call(
        body,
        grid=(n // tn,),
        in_specs=[
            pl.BlockSpec((tn, c), lambda i: (i, 0)),
            pl.BlockSpec((tn, 1), lambda i: (i, 0)),
            pl.BlockSpec(memory_space=pltpu.SMEM),
        ],
        out_specs=pl.BlockSpec((1, 1), lambda i: (0, 0)),
        scratch_shapes=[
            pltpu.VMEM((_N_BINS + 1, c), jnp.float32),
        ],
        out_shape=jax.ShapeDtypeStruct((1, 1), jnp.float32),
        compiler_params=pltpu.CompilerParams(
            dimension_semantics=("arbitrary",)),
    )(logits, lab2, bounds2)
    return out.reshape(-1)
